# baseline (speedup 1.0000x reference)
import functools

import jax
import jax.numpy as jnp
from jax import lax
from jax.experimental import pallas as pl
from jax.experimental.pallas import tpu as pltpu
from jax.experimental.pallas import tpu_sc as plsc

N, E, D, B = 10000, 160000, 256, 16
NP = 10240
EP = 163840
NACC = 11264
JUNK = 10240
CH = 80
KR = 4
SB = 8
HD = D // 2
SP = NP // 16
SPD = NACC // 16
DW = 64
R = 1024
GRID = NP // R

_f32 = jnp.float32
_mesh = plsc.VectorSubcoreMesh(core_axis_name="c", subcore_axis_name="s")


@functools.partial(
    pl.kernel,
    out_type=jax.ShapeDtypeStruct((2 * NACC, DW), _f32),
    mesh=_mesh,
    scratch_types=[
        pltpu.VMEM_SHARED((NACC, DW), _f32),
        pltpu.VMEM((EP // 32 // CH, CH), jnp.int32),
        pltpu.VMEM((CH, DW), _f32),
        pltpu.SemaphoreType.DMA,
        pltpu.SemaphoreType.DMA,
        pltpu.SemaphoreType.DMA,
        pltpu.SemaphoreType.DMA,
    ],
)
def _sc_degree(dst_hbm, ones_hbm, zeros_hbm, cnt_hbm, deg_sh, didx, ones_v,
               sm0, sm1, sm2, sm3):
    c = lax.axis_index("c")
    s = lax.axis_index("s")
    ssem = (sm0, sm1, sm2, sm3)
    pltpu.sync_copy(zeros_hbm.at[pl.ds(s * SPD, SPD)],
                    deg_sh.at[pl.ds(s * SPD, SPD)])
    pltpu.sync_copy(ones_hbm, ones_v)
    cbase = (c * 16 + s) * (EP // 32 // CH)
    pltpu.sync_copy(dst_hbm.at[pl.ds(cbase, EP // 32 // CH)], didx)
    plsc.subcore_barrier()

    @pl.loop(0, EP // 32 // CH, step=4)
    def _(k):
        sd = [pltpu.async_copy(ones_v, deg_sh.at[didx.at[k + j]], ssem[j],
                               add=True) for j in range(4)]
        for j in range(4):
            sd[j].wait()

    plsc.subcore_barrier()
    pltpu.sync_copy(deg_sh.at[pl.ds(s * SPD, SPD)],
                    cnt_hbm.at[pl.ds(c * NACC + s * SPD, SPD)])


@functools.partial(
    pl.kernel,
    out_type=[jax.ShapeDtypeStruct((NP, HD), _f32),
              jax.ShapeDtypeStruct((NP, HD), _f32)],
    mesh=_mesh,
    scratch_types=[
        pltpu.VMEM_SHARED((NP, HD), _f32),
        pltpu.VMEM((SB, CH), jnp.int32),
        pltpu.VMEM((SB, CH), jnp.int32),
        pltpu.VMEM((KR, CH, HD), _f32),
        [pltpu.SemaphoreType.DMA] * KR,
        [pltpu.SemaphoreType.DMA] * KR,
    ],
)
def _sc_scatter(ya_hbm, yb_hbm, src_hbm, dst_hbm, za_hbm, zb_hbm,
                acc, sidx, didx, rows, gsem, ssem):
    c = lax.axis_index("c")
    s = lax.axis_index("s")
    cbase = s * (EP // 16 // CH)

    def run(y_ref):
        pltpu.sync_copy(y_ref.at[pl.ds(s * SP, SP)], acc.at[pl.ds(s * SP, SP)])
        plsc.subcore_barrier()

        @pl.loop(0, EP // 16 // CH, step=SB)
        def _(k):
            pltpu.sync_copy(src_hbm.at[pl.ds(cbase + k, SB)], sidx)
            pltpu.sync_copy(dst_hbm.at[pl.ds(cbase + k, SB)], didx)
            gd, sd = {}, {}
            pair = lambda m: (0, 1) if m % 2 == 0 else (2, 3)
            NB = SB // 2
            for j, b in enumerate(pair(0)):
                gd[(0, j)] = pltpu.async_copy(y_ref.at[sidx.at[j]],
                                              rows.at[b], gsem[b])
            for m in range(NB):
                if m + 1 < NB:
                    for j, b in enumerate(pair(m + 1)):
                        if m >= 1:
                            sd[(m - 1, j)].wait()
                        gd[(m + 1, j)] = pltpu.async_copy(
                            y_ref.at[sidx.at[2 * (m + 1) + j]],
                            rows.at[b], gsem[b])
                for j, b in enumerate(pair(m)):
                    gd[(m, j)].wait()
                    sd[(m, j)] = pltpu.async_copy(
                        rows.at[b], acc.at[didx.at[2 * m + j]],
                        ssem[b], add=True)
            for m in (NB - 2, NB - 1):
                for j in range(2):
                    sd[(m, j)].wait()

    @pl.when(c == 0)
    def _():
        run(ya_hbm)

    @pl.when(c == 1)
    def _():
        run(yb_hbm)

    plsc.subcore_barrier()

    @pl.when(c == 0)
    def _():
        pltpu.sync_copy(acc.at[pl.ds(s * SP, SP)], za_hbm.at[pl.ds(s * SP, SP)])

    @pl.when(c == 1)
    def _():
        pltpu.sync_copy(acc.at[pl.ds(s * SP, SP)], zb_hbm.at[pl.ds(s * SP, SP)])


def _matmul_body(x_ref, w_ref, xa_ref, xb_ref):
    y = jnp.dot(x_ref[...], w_ref[...], preferred_element_type=_f32)
    xa_ref[...] = y[:, :HD]
    xb_ref[...] = y[:, HD:]


_tc_matmul = pl.pallas_call(
    _matmul_body,
    grid=(GRID,),
    in_specs=[
        pl.BlockSpec((R, D), lambda i: (i, 0)),
        pl.BlockSpec((D, D), lambda i: (0, 0)),
    ],
    out_specs=[
        pl.BlockSpec((R, HD), lambda i: (i, 0)),
        pl.BlockSpec((R, HD), lambda i: (i, 0)),
    ],
    out_shape=[
        jax.ShapeDtypeStruct((NP, HD), _f32),
        jax.ShapeDtypeStruct((NP, HD), _f32),
    ],
)


def _scale_body(c0_ref, c1_ref, xa_ref, xb_ref, ya_ref, yb_ref, dinv_ref):
    deg = c0_ref[:, 0:1] + c1_ref[:, 0:1] + 1.0
    row0 = (lax.broadcasted_iota(jnp.int32, (R, 1), 0) == 0).astype(_f32)
    deg = deg - jnp.where(pl.program_id(0) == 0, float(EP - E), 0.0) * row0
    dinv = lax.rsqrt(deg)
    ya_ref[...] = xa_ref[...] * dinv
    yb_ref[...] = xb_ref[...] * dinv
    dinv_ref[...] = dinv


_tc_scale = pl.pallas_call(
    _scale_body,
    grid=(GRID,),
    in_specs=[
        pl.BlockSpec((R, DW), lambda i: (i, 0)),
        pl.BlockSpec((R, DW), lambda i: (NACC // R + i, 0)),
        pl.BlockSpec((R, HD), lambda i: (i, 0)),
        pl.BlockSpec((R, HD), lambda i: (i, 0)),
    ],
    out_specs=[
        pl.BlockSpec((R, HD), lambda i: (i, 0)),
        pl.BlockSpec((R, HD), lambda i: (i, 0)),
        pl.BlockSpec((R, 1), lambda i: (i, 0)),
    ],
    out_shape=[
        jax.ShapeDtypeStruct((NP, HD), _f32),
        jax.ShapeDtypeStruct((NP, HD), _f32),
        jax.ShapeDtypeStruct((NP, 1), _f32),
    ],
)


def _mid_body(za_ref, zb_ref, dinv_ref, b_ref, w_ref, ya_ref, yb_ref):
    z = jnp.concatenate([za_ref[...], zb_ref[...]], axis=1)
    dinv = dinv_ref[...]
    h = jnp.maximum(z * dinv + b_ref[...], 0.0)
    y = jnp.dot(h, w_ref[...], preferred_element_type=_f32) * dinv
    ya_ref[...] = y[:, :HD]
    yb_ref[...] = y[:, HD:]


_tc_mid = pl.pallas_call(
    _mid_body,
    grid=(GRID,),
    in_specs=[
        pl.BlockSpec((R, HD), lambda i: (i, 0)),
        pl.BlockSpec((R, HD), lambda i: (i, 0)),
        pl.BlockSpec((R, 1), lambda i: (i, 0)),
        pl.BlockSpec((1, D), lambda i: (0, 0)),
        pl.BlockSpec((D, D), lambda i: (0, 0)),
    ],
    out_specs=[
        pl.BlockSpec((R, HD), lambda i: (i, 0)),
        pl.BlockSpec((R, HD), lambda i: (i, 0)),
    ],
    out_shape=[
        jax.ShapeDtypeStruct((NP, HD), _f32),
        jax.ShapeDtypeStruct((NP, HD), _f32),
    ],
)


def _final_body(za_ref, zb_ref, dinv_ref, b_ref, bat_ref, out_ref,
                sums_scr, cnts_scr):
    i = pl.program_id(0)
    z = jnp.concatenate([za_ref[...], zb_ref[...]], axis=1)
    h = z * dinv_ref[...] + b_ref[...]
    bat = bat_ref[0, 0, :]
    onehot = (bat[:, None]
              == lax.broadcasted_iota(jnp.int32, (R, B), 1)).astype(_f32)
    blk_sums = lax.dot_general(onehot, h, (((0,), (0,)), ((), ())),
                               preferred_element_type=_f32)
    blk_cnts = jnp.broadcast_to(jnp.sum(onehot, axis=0)[:, None], (B, HD))

    @pl.when(i == 0)
    def _():
        sums_scr[...] = blk_sums
        cnts_scr[...] = blk_cnts

    @pl.when(i > 0)
    def _():
        sums_scr[...] += blk_sums
        cnts_scr[...] += blk_cnts

    @pl.when(i == GRID - 1)
    def _():
        out_ref[...] = sums_scr[...] / jnp.maximum(cnts_scr[:, 0:1], 1.0)


_tc_final = pl.pallas_call(
    _final_body,
    grid=(GRID,),
    in_specs=[
        pl.BlockSpec((R, HD), lambda i: (i, 0)),
        pl.BlockSpec((R, HD), lambda i: (i, 0)),
        pl.BlockSpec((R, 1), lambda i: (i, 0)),
        pl.BlockSpec((1, D), lambda i: (0, 0)),
        pl.BlockSpec((1, 1, R), lambda i: (i, 0, 0)),
    ],
    out_specs=pl.BlockSpec((B, D), lambda i: (0, 0)),
    out_shape=jax.ShapeDtypeStruct((B, D), _f32),
    scratch_shapes=[
        pltpu.VMEM((B, D), _f32),
        pltpu.VMEM((B, HD), _f32),
    ],
)


def kernel(x, edge_index, ptr, batch, W1, b1, W2, b2, W3, b3):
    x = x.astype(_f32)
    xp = jnp.zeros((NP, D), _f32).at[:N].set(x)
    src = edge_index[0]
    dst = edge_index[1]
    srcp = jnp.concatenate(
        [src, jnp.full((EP - E,), N, jnp.int32)]).reshape(EP // CH, CH)
    dstp = jnp.concatenate(
        [dst, jnp.zeros((EP - E,), jnp.int32)]).reshape(EP // CH, CH)
    batp = jnp.concatenate(
        [batch, jnp.full((NP - N,), B, jnp.int32)]).reshape(GRID, 1, R)
    onescol = jnp.zeros((CH, DW), _f32).at[:, 0].set(1.0)
    zeros_acc = jnp.zeros((NACC, DW), _f32)

    cnt = _sc_degree(dstp, onescol, zeros_acc)
    xa, xb = _tc_matmul(xp, W1)
    ya, yb, dinv = _tc_scale(cnt, cnt, xa, xb)
    za, zb = _sc_scatter(ya, yb, srcp, dstp)
    ya, yb = _tc_mid(za, zb, dinv, b1.reshape(1, D), W2)
    za, zb = _sc_scatter(ya, yb, srcp, dstp)
    ya, yb = _tc_mid(za, zb, dinv, b2.reshape(1, D), W3)
    za, zb = _sc_scatter(ya, yb, srcp, dstp)
    mean = _tc_final(za, zb, dinv, b3.reshape(1, D), batp)
    return mean.reshape(-1)


# speedup vs baseline: 1.0747x; 1.0747x over previous
import functools

import jax
import jax.numpy as jnp
from jax import lax
from jax.experimental import pallas as pl
from jax.experimental.pallas import tpu as pltpu
from jax.experimental.pallas import tpu_sc as plsc

N, E, D, B = 10000, 160000, 256, 16
NP = 10240
EP = 163840
NACC = 11264
JUNK = 10240
CH = 80
KR = 4
SB = 8
HD = D // 2
SP = NP // 16
SPD = NACC // 16
DW = 64
R = 1024
GRID = NP // R

_f32 = jnp.float32
_mesh = plsc.VectorSubcoreMesh(core_axis_name="c", subcore_axis_name="s")


@functools.partial(
    pl.kernel,
    out_type=jax.ShapeDtypeStruct((2 * NACC, DW), _f32),
    mesh=_mesh,
    scratch_types=[
        pltpu.VMEM_SHARED((NACC, DW), _f32),
        pltpu.VMEM((EP // 32 // CH, CH), jnp.int32),
        pltpu.VMEM((CH, DW), _f32),
        pltpu.SemaphoreType.DMA,
        pltpu.SemaphoreType.DMA,
        pltpu.SemaphoreType.DMA,
        pltpu.SemaphoreType.DMA,
    ],
)
def _sc_degree(dst_hbm, ones_hbm, zeros_hbm, cnt_hbm, deg_sh, didx, ones_v,
               sm0, sm1, sm2, sm3):
    c = lax.axis_index("c")
    s = lax.axis_index("s")
    ssem = (sm0, sm1, sm2, sm3)
    pltpu.sync_copy(zeros_hbm.at[pl.ds(s * SPD, SPD)],
                    deg_sh.at[pl.ds(s * SPD, SPD)])
    pltpu.sync_copy(ones_hbm, ones_v)
    cbase = (c * 16 + s) * (EP // 32 // CH)
    pltpu.sync_copy(dst_hbm.at[pl.ds(cbase, EP // 32 // CH)], didx)
    plsc.subcore_barrier()

    @pl.loop(0, EP // 32 // CH, step=4)
    def _(k):
        sd = [pltpu.async_copy(ones_v, deg_sh.at[didx.at[k + j]], ssem[j],
                               add=True) for j in range(4)]
        for j in range(4):
            sd[j].wait()

    plsc.subcore_barrier()
    pltpu.sync_copy(deg_sh.at[pl.ds(s * SPD, SPD)],
                    cnt_hbm.at[pl.ds(c * NACC + s * SPD, SPD)])


@functools.partial(
    pl.kernel,
    out_type=[jax.ShapeDtypeStruct((NP, HD), _f32),
              jax.ShapeDtypeStruct((NP, HD), _f32)],
    mesh=_mesh,
    scratch_types=[
        pltpu.VMEM_SHARED((NP, HD), _f32),
        pltpu.VMEM((SB, CH), jnp.int32),
        pltpu.VMEM((SB, CH), jnp.int32),
        pltpu.VMEM((KR, CH, HD), _f32),
        [pltpu.SemaphoreType.DMA] * KR,
        [pltpu.SemaphoreType.DMA] * KR,
    ],
)
def _sc_scatter(ya_hbm, yb_hbm, src_hbm, dst_hbm, za_hbm, zb_hbm,
                acc, sidx, didx, rows, gsem, ssem):
    c = lax.axis_index("c")
    s = lax.axis_index("s")
    cbase = s * (EP // 16 // CH)

    def run(y_ref):
        pltpu.sync_copy(y_ref.at[pl.ds(s * SP, SP)], acc.at[pl.ds(s * SP, SP)])
        plsc.subcore_barrier()

        @pl.loop(0, EP // 16 // CH, step=SB)
        def _(k):
            pltpu.sync_copy(src_hbm.at[pl.ds(cbase + k, SB)], sidx)
            pltpu.sync_copy(dst_hbm.at[pl.ds(cbase + k, SB)], didx)
            gd, sd = {}, {}
            pair = lambda m: (0, 1) if m % 2 == 0 else (2, 3)
            NB = SB // 2
            for j, b in enumerate(pair(0)):
                gd[(0, j)] = pltpu.async_copy(y_ref.at[sidx.at[j]],
                                              rows.at[b], gsem[b])
            for m in range(NB):
                if m + 1 < NB:
                    for j, b in enumerate(pair(m + 1)):
                        if m >= 1:
                            sd[(m - 1, j)].wait()
                        gd[(m + 1, j)] = pltpu.async_copy(
                            y_ref.at[sidx.at[2 * (m + 1) + j]],
                            rows.at[b], gsem[b])
                for j, b in enumerate(pair(m)):
                    gd[(m, j)].wait()
                    sd[(m, j)] = pltpu.async_copy(
                        rows.at[b], acc.at[didx.at[2 * m + j]],
                        ssem[b], add=True)
            for m in (NB - 2, NB - 1):
                for j in range(2):
                    sd[(m, j)].wait()

    @pl.when(c == 0)
    def _():
        run(ya_hbm)

    @pl.when(c == 1)
    def _():
        run(yb_hbm)

    plsc.subcore_barrier()

    @pl.when(c == 0)
    def _():
        pltpu.sync_copy(acc.at[pl.ds(s * SP, SP)], za_hbm.at[pl.ds(s * SP, SP)])

    @pl.when(c == 1)
    def _():
        pltpu.sync_copy(acc.at[pl.ds(s * SP, SP)], zb_hbm.at[pl.ds(s * SP, SP)])


def _prep_body(c0_ref, c1_ref, x_ref, w_ref, ya_ref, yb_ref, dinv_ref):
    deg = c0_ref[:, 0:1] + c1_ref[:, 0:1] + 1.0
    row0 = (lax.broadcasted_iota(jnp.int32, (R, 1), 0) == 0).astype(_f32)
    deg = deg - jnp.where(pl.program_id(0) == 0, float(EP - E), 0.0) * row0
    dinv = lax.rsqrt(deg)
    y = jnp.dot(x_ref[...], w_ref[...], preferred_element_type=_f32) * dinv
    ya_ref[...] = y[:, :HD]
    yb_ref[...] = y[:, HD:]
    dinv_ref[...] = dinv


_tc_prep = pl.pallas_call(
    _prep_body,
    grid=(GRID,),
    in_specs=[
        pl.BlockSpec((R, DW), lambda i: (i, 0)),
        pl.BlockSpec((R, DW), lambda i: (NACC // R + i, 0)),
        pl.BlockSpec((R, D), lambda i: (i, 0)),
        pl.BlockSpec((D, D), lambda i: (0, 0)),
    ],
    out_specs=[
        pl.BlockSpec((R, HD), lambda i: (i, 0)),
        pl.BlockSpec((R, HD), lambda i: (i, 0)),
        pl.BlockSpec((R, 1), lambda i: (i, 0)),
    ],
    out_shape=[
        jax.ShapeDtypeStruct((NP, HD), _f32),
        jax.ShapeDtypeStruct((NP, HD), _f32),
        jax.ShapeDtypeStruct((NP, 1), _f32),
    ],
)


def _mid_body(za_ref, zb_ref, dinv_ref, b_ref, w_ref, ya_ref, yb_ref):
    z = jnp.concatenate([za_ref[...], zb_ref[...]], axis=1)
    dinv = dinv_ref[...]
    h = jnp.maximum(z * dinv + b_ref[...], 0.0)
    y = jnp.dot(h, w_ref[...], preferred_element_type=_f32) * dinv
    ya_ref[...] = y[:, :HD]
    yb_ref[...] = y[:, HD:]


_tc_mid = pl.pallas_call(
    _mid_body,
    grid=(GRID,),
    in_specs=[
        pl.BlockSpec((R, HD), lambda i: (i, 0)),
        pl.BlockSpec((R, HD), lambda i: (i, 0)),
        pl.BlockSpec((R, 1), lambda i: (i, 0)),
        pl.BlockSpec((1, D), lambda i: (0, 0)),
        pl.BlockSpec((D, D), lambda i: (0, 0)),
    ],
    out_specs=[
        pl.BlockSpec((R, HD), lambda i: (i, 0)),
        pl.BlockSpec((R, HD), lambda i: (i, 0)),
    ],
    out_shape=[
        jax.ShapeDtypeStruct((NP, HD), _f32),
        jax.ShapeDtypeStruct((NP, HD), _f32),
    ],
)


def _final_body(za_ref, zb_ref, dinv_ref, b_ref, bat_ref, out_ref,
                sums_scr, cnts_scr):
    i = pl.program_id(0)
    z = jnp.concatenate([za_ref[...], zb_ref[...]], axis=1)
    h = z * dinv_ref[...] + b_ref[...]
    bat = bat_ref[0, 0, :]
    onehot = (bat[:, None]
              == lax.broadcasted_iota(jnp.int32, (R, B), 1)).astype(_f32)
    blk_sums = lax.dot_general(onehot, h, (((0,), (0,)), ((), ())),
                               preferred_element_type=_f32)
    blk_cnts = jnp.broadcast_to(jnp.sum(onehot, axis=0)[:, None], (B, HD))

    @pl.when(i == 0)
    def _():
        sums_scr[...] = blk_sums
        cnts_scr[...] = blk_cnts

    @pl.when(i > 0)
    def _():
        sums_scr[...] += blk_sums
        cnts_scr[...] += blk_cnts

    @pl.when(i == GRID - 1)
    def _():
        out_ref[...] = sums_scr[...] / jnp.maximum(cnts_scr[:, 0:1], 1.0)


_tc_final = pl.pallas_call(
    _final_body,
    grid=(GRID,),
    in_specs=[
        pl.BlockSpec((R, HD), lambda i: (i, 0)),
        pl.BlockSpec((R, HD), lambda i: (i, 0)),
        pl.BlockSpec((R, 1), lambda i: (i, 0)),
        pl.BlockSpec((1, D), lambda i: (0, 0)),
        pl.BlockSpec((1, 1, R), lambda i: (i, 0, 0)),
    ],
    out_specs=pl.BlockSpec((B, D), lambda i: (0, 0)),
    out_shape=jax.ShapeDtypeStruct((B, D), _f32),
    scratch_shapes=[
        pltpu.VMEM((B, D), _f32),
        pltpu.VMEM((B, HD), _f32),
    ],
)


def kernel(x, edge_index, ptr, batch, W1, b1, W2, b2, W3, b3):
    x = x.astype(_f32)
    xp = jnp.zeros((NP, D), _f32).at[:N].set(x)
    src = edge_index[0]
    dst = edge_index[1]
    srcp = jnp.concatenate(
        [src, jnp.full((EP - E,), N, jnp.int32)]).reshape(EP // CH, CH)
    dstp = jnp.concatenate(
        [dst, jnp.zeros((EP - E,), jnp.int32)]).reshape(EP // CH, CH)
    batp = jnp.concatenate(
        [batch, jnp.full((NP - N,), B, jnp.int32)]).reshape(GRID, 1, R)
    onescol = jnp.zeros((CH, DW), _f32).at[:, 0].set(1.0)
    zeros_acc = jnp.zeros((NACC, DW), _f32)

    cnt = _sc_degree(dstp, onescol, zeros_acc)
    ya, yb, dinv = _tc_prep(cnt, cnt, xp, W1)
    za, zb = _sc_scatter(ya, yb, srcp, dstp)
    ya, yb = _tc_mid(za, zb, dinv, b1.reshape(1, D), W2)
    za, zb = _sc_scatter(ya, yb, srcp, dstp)
    ya, yb = _tc_mid(za, zb, dinv, b2.reshape(1, D), W3)
    za, zb = _sc_scatter(ya, yb, srcp, dstp)
    mean = _tc_final(za, zb, dinv, b3.reshape(1, D), batp)
    return mean.reshape(-1)
